# baseline (speedup 1.0000x reference)
import functools

import jax
import jax.numpy as jnp
from jax import lax
from jax.experimental import pallas as pl
from jax.experimental.pallas import tpu as pltpu
from jax.experimental.pallas import tpu_sc as plsc

_NUM_CORES = 2
_NUM_SUBCORES = 16
_LANES = 16
_NW = _NUM_CORES * _NUM_SUBCORES

_BATCH = 16384
_ROW = 128
_ROWS = _BATCH // _ROW
_ROWS_PER_W = _ROWS // _NW


def _sc_gather_sigmoid(ig, idx2d):
    mesh = plsc.VectorSubcoreMesh(
        core_axis_name="c", subcore_axis_name="s", num_cores=_NUM_CORES
    )

    @functools.partial(
        pl.kernel,
        mesh=mesh,
        out_type=jax.ShapeDtypeStruct((_ROWS, _ROW), jnp.float32),
        scratch_types=[
            pltpu.VMEM((_ROWS_PER_W, _ROW), jnp.int32),
            pltpu.VMEM((_ROWS_PER_W, _ROW), jnp.float32),
        ]
        + [pltpu.SemaphoreType.DMA] * (_ROWS_PER_W + 2),
    )
    def body(table_hbm, idx_hbm, out_hbm, idx_v, vals_v, *sems):
        wid = lax.axis_index("s") * _NUM_CORES + lax.axis_index("c")
        base = wid * _ROWS_PER_W
        half = _ROWS_PER_W // 2
        idx_copies = [
            pltpu.async_copy(
                idx_hbm.at[pl.ds(base + h * half, half)],
                idx_v.at[pl.ds(h * half, half)],
                sems[_ROWS_PER_W + h],
            )
            for h in range(2)
        ]
        copies = []
        for h in range(2):
            idx_copies[h].wait()
            for j in range(h * half, (h + 1) * half):
                copies.append(
                    pltpu.async_copy(table_hbm.at[idx_v.at[j]], vals_v.at[j], sems[j])
                )
        out_copies = []
        n_sl = _ROW // _LANES
        for j in range(_ROWS_PER_W):
            copies[j].wait()
            row = vals_v.at[j]
            vs = [row[pl.ds(k * _LANES, _LANES)] for k in range(n_sl)]
            ys = [1.0 / (1.0 + jnp.exp(-v)) for v in vs]
            for k in range(n_sl):
                row[pl.ds(k * _LANES, _LANES)] = ys[k]
            out_copies.append(pltpu.async_copy(row, out_hbm.at[base + j], sems[j]))
        for c in out_copies:
            c.wait()

    return body(ig, idx2d)


def kernel(indices, ig):
    idx2d = indices.reshape(_ROWS, _ROW)
    out = _sc_gather_sigmoid(ig, idx2d)
    return out.reshape(_BATCH)


# speedup vs baseline: 1.0154x; 1.0154x over previous
import functools

import jax
import jax.numpy as jnp
from jax import lax
from jax.experimental import pallas as pl
from jax.experimental.pallas import tpu as pltpu
from jax.experimental.pallas import tpu_sc as plsc

_NUM_CORES = 1
_NUM_SUBCORES = 16
_LANES = 16
_NW = _NUM_CORES * _NUM_SUBCORES

_BATCH = 16384
_ROW = 128
_ROWS = _BATCH // _ROW
_ROWS_PER_W = _ROWS // _NW


def _sc_gather_sigmoid(ig, idx2d):
    mesh = plsc.VectorSubcoreMesh(
        core_axis_name="c", subcore_axis_name="s", num_cores=_NUM_CORES
    )

    @functools.partial(
        pl.kernel,
        mesh=mesh,
        out_type=jax.ShapeDtypeStruct((_ROWS, _ROW), jnp.float32),
        scratch_types=[
            pltpu.VMEM((_ROWS_PER_W, _ROW), jnp.int32),
            pltpu.VMEM((_ROWS_PER_W, _ROW), jnp.float32),
        ]
        + [pltpu.SemaphoreType.DMA] * (_ROWS_PER_W + 2),
    )
    def body(table_hbm, idx_hbm, out_hbm, idx_v, vals_v, *sems):
        wid = lax.axis_index("s") * _NUM_CORES + lax.axis_index("c")
        base = wid * _ROWS_PER_W
        half = _ROWS_PER_W // 2
        idx_copies = [
            pltpu.async_copy(
                idx_hbm.at[pl.ds(base + h * half, half)],
                idx_v.at[pl.ds(h * half, half)],
                sems[_ROWS_PER_W + h],
            )
            for h in range(2)
        ]
        copies = []
        for h in range(2):
            idx_copies[h].wait()
            for j in range(h * half, (h + 1) * half):
                copies.append(
                    pltpu.async_copy(table_hbm.at[idx_v.at[j]], vals_v.at[j], sems[j])
                )
        out_copies = []
        n_sl = _ROW // _LANES
        for j in range(_ROWS_PER_W):
            copies[j].wait()
            row = vals_v.at[j]
            vs = [row[pl.ds(k * _LANES, _LANES)] for k in range(n_sl)]
            ys = [1.0 / (1.0 + jnp.exp(-v)) for v in vs]
            for k in range(n_sl):
                row[pl.ds(k * _LANES, _LANES)] = ys[k]
            out_copies.append(pltpu.async_copy(row, out_hbm.at[base + j], sems[j]))
        for c in out_copies:
            c.wait()

    return body(ig, idx2d)


def kernel(indices, ig):
    idx2d = indices.reshape(_ROWS, _ROW)
    out = _sc_gather_sigmoid(ig, idx2d)
    return out.reshape(_BATCH)
